# Initial kernel scaffold; baseline (speedup 1.0000x reference)
import jax
import jax.numpy as jnp
from jax.experimental import pallas as pl


def kernel(x, pos, edge_index, embed_W, embed_b, edge_W1, edge_b1, edge_W2, edge_b2, node_W1, node_b1, node_W2, node_b2, coord_W1, coord_b1, coord_W2, out_W1, out_b1, out_W2, out_b2):
    raise NotImplementedError("write your pallas kernel here")


# speedup vs baseline: 2.5286x; 2.5286x over previous
import functools

import jax
import jax.numpy as jnp
from jax import lax
from jax.experimental import pallas as pl
from jax.experimental.pallas import tpu as pltpu
from jax.experimental.pallas import tpu_sc as plsc

N = 10000
E = 320000
D = 128
P = 16
W = 2 * D
DA = D + P
NC = 2
NS = 16
NW = NC * NS
EW = E // NW

CHG = 80
NCHG = EW // CHG
CHS = 80
NCHS = EW // CHS

RPT = 624
REM0 = NS * RPT
REM = N - REM0


def _sc_gather_body(hp_hbm, src_hbm, dst_hbm,
                    hps_hbm, hpd_hbm,
                    idx_v, row_v, sem):
    c = lax.axis_index("c")
    s = lax.axis_index("s")
    wid = s * NC + c

    def body(i, carry):
        base = wid * EW + i * CHG
        pltpu.sync_copy(src_hbm.at[pl.ds(base, CHG)], idx_v)
        pltpu.async_copy(hp_hbm.at[idx_v], row_v, sem).wait()
        pltpu.sync_copy(row_v, hps_hbm.at[pl.ds(base, CHG)])
        pltpu.sync_copy(dst_hbm.at[pl.ds(base, CHG)], idx_v)
        pltpu.async_copy(hp_hbm.at[idx_v], row_v, sem).wait()
        pltpu.sync_copy(row_v, hpd_hbm.at[pl.ds(base, CHG)])
        return carry

    lax.fori_loop(0, NCHG, body, 0)


def _sc_scatter_body(ef_hbm, aux_hbm, dst_hbm, z_hbm,
                     agg0_hbm, agg1_hbm, aux0_hbm, aux1_hbm,
                     idx_v, vval, sh):
    c = lax.axis_index("c")
    s = lax.axis_index("s")
    wid = s * NC + c
    row0 = s * RPT

    def _zero():
        pltpu.sync_copy(z_hbm.at[pl.ds(row0, RPT)], sh.at[pl.ds(row0, RPT)])

        @pl.when(s == NS - 1)
        def _():
            pltpu.sync_copy(z_hbm.at[pl.ds(REM0, REM)], sh.at[pl.ds(REM0, REM)])

    def _accumulate(val_hbm):
        def body(i, carry):
            base = wid * EW + i * CHS
            pltpu.sync_copy(dst_hbm.at[pl.ds(base, CHS)], idx_v)
            pltpu.sync_copy(val_hbm.at[pl.ds(base, CHS)], vval)
            pltpu.sync_copy(vval, sh.at[idx_v], add=True)
            return carry
        lax.fori_loop(0, NCHS, body, 0)

    def _writeout(o0_hbm, o1_hbm):
        @pl.when(c == 0)
        def _():
            pltpu.sync_copy(sh.at[pl.ds(row0, RPT)], o0_hbm.at[pl.ds(row0, RPT)])

            @pl.when(s == NS - 1)
            def _():
                pltpu.sync_copy(sh.at[pl.ds(REM0, REM)], o0_hbm.at[pl.ds(REM0, REM)])

        @pl.when(c == 1)
        def _():
            pltpu.sync_copy(sh.at[pl.ds(row0, RPT)], o1_hbm.at[pl.ds(row0, RPT)])

            @pl.when(s == NS - 1)
            def _():
                pltpu.sync_copy(sh.at[pl.ds(REM0, REM)], o1_hbm.at[pl.ds(REM0, REM)])

    _zero()
    plsc.subcore_barrier()
    _accumulate(ef_hbm)
    plsc.subcore_barrier()
    _writeout(agg0_hbm, agg1_hbm)
    _zero()
    plsc.subcore_barrier()
    _accumulate(aux_hbm)
    plsc.subcore_barrier()
    _writeout(aux0_hbm, aux1_hbm)


@functools.lru_cache(maxsize=None)
def _sc_kernels():
    mesh = plsc.VectorSubcoreMesh(core_axis_name="c", subcore_axis_name="s")
    gather = pl.kernel(
        _sc_gather_body,
        out_type=[
            jax.ShapeDtypeStruct((E, W), jnp.float32),
            jax.ShapeDtypeStruct((E, W), jnp.float32),
        ],
        mesh=mesh,
        scratch_types=[
            pltpu.VMEM((CHG,), jnp.int32),
            pltpu.VMEM((CHG, W), jnp.float32),
            pltpu.SemaphoreType.DMA,
        ],
    )
    scatter = pl.kernel(
        _sc_scatter_body,
        out_type=[
            jax.ShapeDtypeStruct((N, D), jnp.float32),
            jax.ShapeDtypeStruct((N, D), jnp.float32),
            jax.ShapeDtypeStruct((N, D), jnp.float32),
            jax.ShapeDtypeStruct((N, D), jnp.float32),
        ],
        mesh=mesh,
        scratch_types=[
            pltpu.VMEM((CHS,), jnp.int32),
            pltpu.VMEM((CHS, D), jnp.float32),
            pltpu.VMEM_SHARED((N, D), jnp.float32),
        ],
    )
    return gather, scatter


def _silu(x):
    return x * jax.nn.sigmoid(x)


EB = 2000
NB = 2000

_row = lambda i: (i, 0)
_row1 = lambda i: (i, 1)
_fix = lambda i: (0, 0)


def _edge_phase(hps, hpd, w1, b1, w2, b2, cw1, cb1, cw2):
    def body(hs_ref, ps_ref, hd_ref, pd_ref, w1_ref,
             b1_ref, w2_ref, b2_ref, cw1_ref, cb1_ref, cw2_ref,
             ef_ref, aux_ref):
        ev = pd_ref[:, :P] - ps_ref[:, :P]
        dist = jnp.sum(ev * ev, axis=1, keepdims=True)
        e_in = jnp.concatenate([hs_ref[...], hd_ref[...], dist], axis=1)
        t = e_in @ w1_ref[...] + b1_ref[...]
        ef = _silu(_silu(t) @ w2_ref[...] + b2_ref[...])
        v = _silu(ef @ cw1_ref[...] + cb1_ref[...])
        cw = v @ cw2_ref[...]
        ef_ref[...] = ef
        lane = lax.broadcasted_iota(jnp.int32, (EB, D), 1)
        evw = jnp.pad(ev, ((0, 0), (0, D - P))) * cw
        aux_ref[...] = jnp.where(lane == P - 1, 1.0, evw)

    return pl.pallas_call(
        body,
        grid=(E // EB,),
        in_specs=[
            pl.BlockSpec((EB, D), _row),
            pl.BlockSpec((EB, D), _row1),
            pl.BlockSpec((EB, D), _row),
            pl.BlockSpec((EB, D), _row1),
            pl.BlockSpec((2 * D + 1, D), _fix),
            pl.BlockSpec((1, D), _fix),
            pl.BlockSpec((D, D), _fix),
            pl.BlockSpec((1, D), _fix),
            pl.BlockSpec((D, D), _fix),
            pl.BlockSpec((1, D), _fix),
            pl.BlockSpec((D, 1), _fix),
        ],
        out_specs=[
            pl.BlockSpec((EB, D), _row),
            pl.BlockSpec((EB, D), _row),
        ],
        out_shape=[
            jax.ShapeDtypeStruct((E, D), jnp.float32),
            jax.ShapeDtypeStruct((E, D), jnp.float32),
        ],
    )(hps, hps, hpd, hpd, w1, b1, w2, b2, cw1, cb1, cw2)


def _node_phase(hp, agg0, agg1, aux0, aux1, nw1, nb1, nw2, nb2):
    def body(h_ref, p_ref, agg0_ref, agg1_ref, aux0_ref, aux1_ref,
             w1_ref, b1_ref, w2_ref, b2_ref, o_ref):
        h = h_ref[...]
        agg = agg0_ref[...] + agg1_ref[...]
        na = jnp.concatenate([h, agg], axis=1)
        t = _silu(na @ w1_ref[...] + b1_ref[...])
        h_new = h + t @ w2_ref[...] + b2_ref[...]
        aux = aux0_ref[...] + aux1_ref[...]
        deg = jnp.maximum(aux[:, P - 1:P], 1.0)
        lane = lax.broadcasted_iota(jnp.int32, (NB, D), 1)
        p_new = p_ref[...] + jnp.where(lane < 3, aux, 0.0) / deg
        o_ref[...] = jnp.concatenate([h_new, p_new], axis=1)

    return pl.pallas_call(
        body,
        grid=(N // NB,),
        in_specs=[
            pl.BlockSpec((NB, D), _row),
            pl.BlockSpec((NB, D), _row1),
            pl.BlockSpec((NB, D), _row),
            pl.BlockSpec((NB, D), _row),
            pl.BlockSpec((NB, D), _row),
            pl.BlockSpec((NB, D), _row),
            pl.BlockSpec((2 * D, D), _fix),
            pl.BlockSpec((1, D), _fix),
            pl.BlockSpec((D, D), _fix),
            pl.BlockSpec((1, D), _fix),
        ],
        out_specs=pl.BlockSpec((NB, W), _row),
        out_shape=jax.ShapeDtypeStruct((N, W), jnp.float32),
    )(hp, hp, agg0, agg1, aux0, aux1, nw1, nb1, nw2, nb2)


def kernel(x, pos, edge_index, embed_W, embed_b, edge_W1, edge_b1, edge_W2,
           edge_b2, node_W1, node_b1, node_W2, node_b2, coord_W1, coord_b1,
           coord_W2, out_W1, out_b1, out_W2, out_b2):
    L = edge_W1.shape[0]
    src = edge_index[0]
    dst = edge_index[1]
    p128 = jnp.pad(pos, ((0, 0), (0, D - 3)))
    z128 = jnp.zeros((N, D), jnp.float32)

    def embed_body(x_ref, p_ref, w_ref, b_ref, o_ref):
        h = _silu(x_ref[...] @ w_ref[...] + b_ref[...])
        o_ref[...] = jnp.concatenate([h, p_ref[...]], axis=1)
    hp = pl.pallas_call(
        embed_body,
        grid=(N // NB,),
        in_specs=[
            pl.BlockSpec((NB, D), _row),
            pl.BlockSpec((NB, D), _row),
            pl.BlockSpec((D, D), _fix),
            pl.BlockSpec((1, D), _fix),
        ],
        out_specs=pl.BlockSpec((NB, W), _row),
        out_shape=jax.ShapeDtypeStruct((N, W), jnp.float32),
    )(x, p128, embed_W, embed_b[None])

    sc_gather, sc_scatter = _sc_kernels()
    for l in range(L):
        hps, hpd = sc_gather(hp, src, dst)
        ef, aux = _edge_phase(
            hps, hpd, edge_W1[l], edge_b1[l][None], edge_W2[l],
            edge_b2[l][None], coord_W1[l], coord_b1[l][None], coord_W2[l])
        agg0, agg1, aux0, aux1 = sc_scatter(ef, aux, dst, z128)
        hp = _node_phase(hp, agg0, agg1, aux0, aux1,
                         node_W1[l], node_b1[l][None],
                         node_W2[l], node_b2[l][None])

    def out_body(h_ref, w1_ref, b1_ref, w2_ref, b2_ref, o_ref):
        t = _silu(h_ref[...] @ w1_ref[...] + b1_ref[...])
        o_ref[...] = t @ w2_ref[...] + b2_ref[...]
    out = pl.pallas_call(
        out_body,
        grid=(N // NB,),
        in_specs=[
            pl.BlockSpec((NB, D), _row),
            pl.BlockSpec((D, D), _fix),
            pl.BlockSpec((1, D), _fix),
            pl.BlockSpec((D, D), _fix),
            pl.BlockSpec((1, D), _fix),
        ],
        out_specs=pl.BlockSpec((NB, D), _row),
        out_shape=jax.ShapeDtypeStruct((N, D), jnp.float32),
    )(hp, out_W1, out_b1[None], out_W2, out_b2[None])

    return (out, hp[:, D:D + 3])


# speedup vs baseline: 2.7486x; 1.0870x over previous
import functools

import jax
import jax.numpy as jnp
from jax import lax
from jax.experimental import pallas as pl
from jax.experimental.pallas import tpu as pltpu
from jax.experimental.pallas import tpu_sc as plsc

N = 10000
E = 320000
D = 128
P = 16
W = 2 * D
DA = D + P
NC = 2
NS = 16
NW = NC * NS
EW = E // NW

CHG = 80
NCHG = EW // CHG
CHS = 80
NCHS = EW // CHS

RPT = 624
REM0 = NS * RPT
REM = N - REM0


def _sc_gather_body(hp_hbm, src_hbm, dst_hbm,
                    hps_hbm, hpd_hbm,
                    idxs_v, idxd_v, rows_v, rowd_v, sems, semd):
    c = lax.axis_index("c")
    s = lax.axis_index("s")
    wid = s * NC + c

    def body(i, carry):
        base = wid * EW + i * CHG
        pltpu.sync_copy(src_hbm.at[pl.ds(base, CHG)], idxs_v)
        pltpu.sync_copy(dst_hbm.at[pl.ds(base, CHG)], idxd_v)
        cps = pltpu.async_copy(hp_hbm.at[idxs_v], rows_v, sems)
        cpd = pltpu.async_copy(hp_hbm.at[idxd_v], rowd_v, semd)
        cps.wait()
        pltpu.sync_copy(rows_v, hps_hbm.at[pl.ds(base, CHG)])
        cpd.wait()
        pltpu.sync_copy(rowd_v, hpd_hbm.at[pl.ds(base, CHG)])
        return carry

    lax.fori_loop(0, NCHG, body, 0)


def _sc_scatter_body(ef_hbm, aux_hbm, dst_hbm, z_hbm,
                     agg0_hbm, agg1_hbm, aux0_hbm, aux1_hbm,
                     idx_v, vval, sh):
    c = lax.axis_index("c")
    s = lax.axis_index("s")
    wid = s * NC + c
    row0 = s * RPT

    def _zero():
        pltpu.sync_copy(z_hbm.at[pl.ds(row0, RPT)], sh.at[pl.ds(row0, RPT)])

        @pl.when(s == NS - 1)
        def _():
            pltpu.sync_copy(z_hbm.at[pl.ds(REM0, REM)], sh.at[pl.ds(REM0, REM)])

    def _accumulate(val_hbm):
        def body(i, carry):
            base = wid * EW + i * CHS
            pltpu.sync_copy(dst_hbm.at[pl.ds(base, CHS)], idx_v)
            pltpu.sync_copy(val_hbm.at[pl.ds(base, CHS)], vval)
            pltpu.sync_copy(vval, sh.at[idx_v], add=True)
            return carry
        lax.fori_loop(0, NCHS, body, 0)

    def _writeout(o0_hbm, o1_hbm):
        @pl.when(c == 0)
        def _():
            pltpu.sync_copy(sh.at[pl.ds(row0, RPT)], o0_hbm.at[pl.ds(row0, RPT)])

            @pl.when(s == NS - 1)
            def _():
                pltpu.sync_copy(sh.at[pl.ds(REM0, REM)], o0_hbm.at[pl.ds(REM0, REM)])

        @pl.when(c == 1)
        def _():
            pltpu.sync_copy(sh.at[pl.ds(row0, RPT)], o1_hbm.at[pl.ds(row0, RPT)])

            @pl.when(s == NS - 1)
            def _():
                pltpu.sync_copy(sh.at[pl.ds(REM0, REM)], o1_hbm.at[pl.ds(REM0, REM)])

    _zero()
    plsc.subcore_barrier()
    _accumulate(ef_hbm)
    plsc.subcore_barrier()
    _writeout(agg0_hbm, agg1_hbm)
    _zero()
    plsc.subcore_barrier()
    _accumulate(aux_hbm)
    plsc.subcore_barrier()
    _writeout(aux0_hbm, aux1_hbm)


@functools.lru_cache(maxsize=None)
def _sc_kernels():
    mesh = plsc.VectorSubcoreMesh(core_axis_name="c", subcore_axis_name="s")
    gather = pl.kernel(
        _sc_gather_body,
        out_type=[
            jax.ShapeDtypeStruct((E, W), jnp.float32),
            jax.ShapeDtypeStruct((E, W), jnp.float32),
        ],
        mesh=mesh,
        scratch_types=[
            pltpu.VMEM((CHG,), jnp.int32),
            pltpu.VMEM((CHG,), jnp.int32),
            pltpu.VMEM((CHG, W), jnp.float32),
            pltpu.VMEM((CHG, W), jnp.float32),
            pltpu.SemaphoreType.DMA,
            pltpu.SemaphoreType.DMA,
        ],
    )
    scatter = pl.kernel(
        _sc_scatter_body,
        out_type=[
            jax.ShapeDtypeStruct((N, D), jnp.float32),
            jax.ShapeDtypeStruct((N, D), jnp.float32),
            jax.ShapeDtypeStruct((N, D), jnp.float32),
            jax.ShapeDtypeStruct((N, D), jnp.float32),
        ],
        mesh=mesh,
        scratch_types=[
            pltpu.VMEM((CHS,), jnp.int32),
            pltpu.VMEM((CHS, D), jnp.float32),
            pltpu.VMEM_SHARED((N, D), jnp.float32),
        ],
    )
    return gather, scatter


def _silu(x):
    return x * jax.nn.sigmoid(x)


EB = 2000
NB = 2000

_row = lambda i: (i, 0)
_row1 = lambda i: (i, 1)
_fix = lambda i: (0, 0)


def _edge_phase(hps, hpd, w1, b1, w2, b2, cw1, cb1, cw2):
    def body(hs_ref, ps_ref, hd_ref, pd_ref, w1_ref,
             b1_ref, w2_ref, b2_ref, cw1_ref, cb1_ref, cw2_ref,
             ef_ref, aux_ref):
        ev = pd_ref[:, :P] - ps_ref[:, :P]
        dist = jnp.sum(ev * ev, axis=1, keepdims=True)
        e_in = jnp.concatenate([hs_ref[...], hd_ref[...], dist], axis=1)
        t = e_in @ w1_ref[...] + b1_ref[...]
        ef = _silu(_silu(t) @ w2_ref[...] + b2_ref[...])
        v = _silu(ef @ cw1_ref[...] + cb1_ref[...])
        cw = v @ cw2_ref[...]
        ef_ref[...] = ef
        lane = lax.broadcasted_iota(jnp.int32, (EB, D), 1)
        evw = jnp.pad(ev, ((0, 0), (0, D - P))) * cw
        aux_ref[...] = jnp.where(lane == P - 1, 1.0, evw)

    return pl.pallas_call(
        body,
        grid=(E // EB,),
        in_specs=[
            pl.BlockSpec((EB, D), _row),
            pl.BlockSpec((EB, D), _row1),
            pl.BlockSpec((EB, D), _row),
            pl.BlockSpec((EB, D), _row1),
            pl.BlockSpec((2 * D + 1, D), _fix),
            pl.BlockSpec((1, D), _fix),
            pl.BlockSpec((D, D), _fix),
            pl.BlockSpec((1, D), _fix),
            pl.BlockSpec((D, D), _fix),
            pl.BlockSpec((1, D), _fix),
            pl.BlockSpec((D, 1), _fix),
        ],
        out_specs=[
            pl.BlockSpec((EB, D), _row),
            pl.BlockSpec((EB, D), _row),
        ],
        out_shape=[
            jax.ShapeDtypeStruct((E, D), jnp.float32),
            jax.ShapeDtypeStruct((E, D), jnp.float32),
        ],
    )(hps, hps, hpd, hpd, w1, b1, w2, b2, cw1, cb1, cw2)


def _node_phase(hp, agg0, agg1, aux0, aux1, nw1, nb1, nw2, nb2):
    def body(h_ref, p_ref, agg0_ref, agg1_ref, aux0_ref, aux1_ref,
             w1_ref, b1_ref, w2_ref, b2_ref, o_ref):
        h = h_ref[...]
        agg = agg0_ref[...] + agg1_ref[...]
        na = jnp.concatenate([h, agg], axis=1)
        t = _silu(na @ w1_ref[...] + b1_ref[...])
        h_new = h + t @ w2_ref[...] + b2_ref[...]
        aux = aux0_ref[...] + aux1_ref[...]
        deg = jnp.maximum(aux[:, P - 1:P], 1.0)
        lane = lax.broadcasted_iota(jnp.int32, (NB, D), 1)
        p_new = p_ref[...] + jnp.where(lane < 3, aux, 0.0) / deg
        o_ref[...] = jnp.concatenate([h_new, p_new], axis=1)

    return pl.pallas_call(
        body,
        grid=(N // NB,),
        in_specs=[
            pl.BlockSpec((NB, D), _row),
            pl.BlockSpec((NB, D), _row1),
            pl.BlockSpec((NB, D), _row),
            pl.BlockSpec((NB, D), _row),
            pl.BlockSpec((NB, D), _row),
            pl.BlockSpec((NB, D), _row),
            pl.BlockSpec((2 * D, D), _fix),
            pl.BlockSpec((1, D), _fix),
            pl.BlockSpec((D, D), _fix),
            pl.BlockSpec((1, D), _fix),
        ],
        out_specs=pl.BlockSpec((NB, W), _row),
        out_shape=jax.ShapeDtypeStruct((N, W), jnp.float32),
    )(hp, hp, agg0, agg1, aux0, aux1, nw1, nb1, nw2, nb2)


def kernel(x, pos, edge_index, embed_W, embed_b, edge_W1, edge_b1, edge_W2,
           edge_b2, node_W1, node_b1, node_W2, node_b2, coord_W1, coord_b1,
           coord_W2, out_W1, out_b1, out_W2, out_b2):
    L = edge_W1.shape[0]
    src = edge_index[0]
    dst = edge_index[1]
    p128 = jnp.pad(pos, ((0, 0), (0, D - 3)))
    z128 = jnp.zeros((N, D), jnp.float32)

    def embed_body(x_ref, p_ref, w_ref, b_ref, o_ref):
        h = _silu(x_ref[...] @ w_ref[...] + b_ref[...])
        o_ref[...] = jnp.concatenate([h, p_ref[...]], axis=1)
    hp = pl.pallas_call(
        embed_body,
        grid=(N // NB,),
        in_specs=[
            pl.BlockSpec((NB, D), _row),
            pl.BlockSpec((NB, D), _row),
            pl.BlockSpec((D, D), _fix),
            pl.BlockSpec((1, D), _fix),
        ],
        out_specs=pl.BlockSpec((NB, W), _row),
        out_shape=jax.ShapeDtypeStruct((N, W), jnp.float32),
    )(x, p128, embed_W, embed_b[None])

    sc_gather, sc_scatter = _sc_kernels()
    for l in range(L):
        hps, hpd = sc_gather(hp, src, dst)
        ef, aux = _edge_phase(
            hps, hpd, edge_W1[l], edge_b1[l][None], edge_W2[l],
            edge_b2[l][None], coord_W1[l], coord_b1[l][None], coord_W2[l])
        agg0, agg1, aux0, aux1 = sc_scatter(ef, aux, dst, z128)
        hp = _node_phase(hp, agg0, agg1, aux0, aux1,
                         node_W1[l], node_b1[l][None],
                         node_W2[l], node_b2[l][None])

    def out_body(h_ref, w1_ref, b1_ref, w2_ref, b2_ref, o_ref):
        t = _silu(h_ref[...] @ w1_ref[...] + b1_ref[...])
        o_ref[...] = t @ w2_ref[...] + b2_ref[...]
    out = pl.pallas_call(
        out_body,
        grid=(N // NB,),
        in_specs=[
            pl.BlockSpec((NB, D), _row),
            pl.BlockSpec((D, D), _fix),
            pl.BlockSpec((1, D), _fix),
            pl.BlockSpec((D, D), _fix),
            pl.BlockSpec((1, D), _fix),
        ],
        out_specs=pl.BlockSpec((NB, D), _row),
        out_shape=jax.ShapeDtypeStruct((N, D), jnp.float32),
    )(hp, out_W1, out_b1[None], out_W2, out_b2[None])

    return (out, hp[:, D:D + 3])
